# baseline (speedup 1.0000x reference)
import jax
import jax.numpy as jnp
from jax.experimental import pallas as pl

_R = (0, 46, 167, 100, 191, 220, 0, 10)
_G = (160, 141, 0, 62, 30, 87, 166, 91)
_B = (177, 239, 174, 191, 75, 46, 0, 196)
_PACKED_LUT = tuple(r | (g << 8) | (b << 16) for r, g, b in zip(_R, _G, _B))

_TILE = 2048


def _lsh_kernel(rot_ref, x_ref, perm_ref, r_ref, g_ref, b_ref):
    t = x_ref.shape[1]
    tg = t // 8
    v = jax.lax.dot_general(
        rot_ref[...], x_ref[...],
        dimension_numbers=(((1,), (0,)), ((), ())),
        preferred_element_type=jnp.float32,
    )
    best = v[0:16, :]
    code = jnp.zeros(best.shape, dtype=jnp.float32)
    for k in range(1, 8):
        s = v[16 * k:16 * (k + 1), :]
        gt = s > best
        code = jnp.where(gt, jnp.float32(k), code)
        best = jnp.maximum(best, s)
    codep = jax.lax.dot_general(
        code.astype(jnp.bfloat16), perm_ref[...],
        dimension_numbers=(((1,), (0,)), ((), ())),
        preferred_element_type=jnp.float32,
    )
    stacked = jnp.concatenate(
        [codep[:, p * tg:(p + 1) * tg] for p in range(8)], axis=0
    )
    ci = stacked.T.astype(jnp.int32)
    packed = jnp.full(ci.shape, _PACKED_LUT[0], dtype=jnp.int32)
    for k in range(1, 8):
        packed = jnp.where(ci == k, _PACKED_LUT[k], packed)
    r_ref[...] = (packed & 0xFF).astype(jnp.uint8)
    g_ref[...] = ((packed >> 8) & 0xFF).astype(jnp.uint8)
    b_ref[...] = ((packed >> 16) & 0xFF).astype(jnp.uint8)


def kernel(inp, rotations):
    B, C, H, W = inp.shape
    HW = H * W
    n_hashes = rotations.shape[1]
    x = inp.reshape(C, HW)
    rot = jnp.transpose(rotations, (0, 2, 1)).reshape(C, 4 * n_hashes)
    rot_packed = jnp.concatenate([rot, -rot], axis=1).T

    tg = _TILE // 8
    j = jnp.arange(_TILE, dtype=jnp.int32)
    c_of_j = 8 * (j % tg) + j // tg
    perm = (jnp.arange(_TILE, dtype=jnp.int32)[:, None] == c_of_j[None, :]
            ).astype(jnp.bfloat16)

    out_sds = jax.ShapeDtypeStruct((HW // 8, 128), jnp.uint8)
    r, g, b = pl.pallas_call(
        _lsh_kernel,
        grid=(HW // _TILE,),
        in_specs=[
            pl.BlockSpec((128, C), lambda i: (0, 0)),
            pl.BlockSpec((C, _TILE), lambda i: (0, i)),
            pl.BlockSpec((_TILE, _TILE), lambda i: (0, 0)),
        ],
        out_specs=[pl.BlockSpec((_TILE // 8, 128), lambda i: (i, 0))] * 3,
        out_shape=[out_sds, out_sds, out_sds],
    )(rot_packed, x, perm)
    shape = (B, H, W, n_hashes)
    return (r.reshape(shape), g.reshape(shape), b.reshape(shape))


# speedup vs baseline: 1.4108x; 1.4108x over previous
import jax
import jax.numpy as jnp
from jax.experimental import pallas as pl

_R = (0, 46, 167, 100, 191, 220, 0, 10)
_G = (160, 141, 0, 62, 30, 87, 166, 91)
_B = (177, 239, 174, 191, 75, 46, 0, 196)
_PACKED_LUT = tuple(r | (g << 8) | (b << 16) for r, g, b in zip(_R, _G, _B))

_HBLK = 8


def _lsh_kernel(rot_ref, x_ref, perm_ref, r_ref, g_ref, b_ref):
    w = x_ref.shape[3]
    tg = w // 8
    for row in range(x_ref.shape[2]):
        v = jax.lax.dot_general(
            rot_ref[...], x_ref[0, :, row, :],
            dimension_numbers=(((1,), (0,)), ((), ())),
            preferred_element_type=jnp.float32,
        )
        best = v[0:16, :]
        code = jnp.zeros(best.shape, dtype=jnp.float32)
        for k in range(1, 8):
            s = v[16 * k:16 * (k + 1), :]
            gt = s > best
            code = jnp.where(gt, jnp.float32(k), code)
            best = jnp.maximum(best, s)
        codep = jax.lax.dot_general(
            code.astype(jnp.bfloat16), perm_ref[...],
            dimension_numbers=(((1,), (0,)), ((), ())),
            preferred_element_type=jnp.float32,
        )
        stacked = jnp.concatenate(
            [codep[:, p * tg:(p + 1) * tg] for p in range(8)], axis=0
        )
        ci = stacked.T.astype(jnp.int32)
        packed = jnp.full(ci.shape, _PACKED_LUT[0], dtype=jnp.int32)
        for k in range(1, 8):
            packed = jnp.where(ci == k, _PACKED_LUT[k], packed)
        rows = pl.ds(row * tg, tg)
        r_ref[rows, :] = (packed & 0xFF).astype(jnp.uint8)
        g_ref[rows, :] = ((packed >> 8) & 0xFF).astype(jnp.uint8)
        b_ref[rows, :] = ((packed >> 16) & 0xFF).astype(jnp.uint8)


def kernel(inp, rotations):
    B, C, H, W = inp.shape
    HW = H * W
    n_hashes = rotations.shape[1]
    rot = jnp.transpose(rotations, (0, 2, 1)).reshape(C, 4 * n_hashes)
    rot_packed = jnp.concatenate([rot, -rot], axis=1).T

    tg = W // 8
    j = jnp.arange(W, dtype=jnp.int32)
    c_of_j = 8 * (j % tg) + j // tg
    perm = (jnp.arange(W, dtype=jnp.int32)[:, None] == c_of_j[None, :]
            ).astype(jnp.bfloat16)

    out_sds = jax.ShapeDtypeStruct((HW // 8, 128), jnp.uint8)
    blk_rows = _HBLK * tg
    r, g, b = pl.pallas_call(
        _lsh_kernel,
        grid=(H // _HBLK,),
        in_specs=[
            pl.BlockSpec((128, C), lambda i: (0, 0)),
            pl.BlockSpec((1, C, _HBLK, W), lambda i: (0, 0, i, 0)),
            pl.BlockSpec((W, W), lambda i: (0, 0)),
        ],
        out_specs=[pl.BlockSpec((blk_rows, 128), lambda i: (i, 0))] * 3,
        out_shape=[out_sds, out_sds, out_sds],
    )(rot_packed, inp, perm)
    shape = (B, H, W, n_hashes)
    return (r.reshape(shape), g.reshape(shape), b.reshape(shape))


# speedup vs baseline: 1.4204x; 1.0068x over previous
import jax
import jax.numpy as jnp
from jax.experimental import pallas as pl

_R = (0, 46, 167, 100, 191, 220, 0, 10)
_G = (160, 141, 0, 62, 30, 87, 166, 91)
_B = (177, 239, 174, 191, 75, 46, 0, 196)
_PACKED_LUT = tuple(r | (g << 8) | (b << 16) for r, g, b in zip(_R, _G, _B))

_HBLK = 16


def _lsh_kernel(rot_ref, x_ref, perm_ref, r_ref, g_ref, b_ref):
    w = x_ref.shape[3]
    tg = w // 8
    for row in range(x_ref.shape[2]):
        v = jax.lax.dot_general(
            rot_ref[...], x_ref[0, :, row, :],
            dimension_numbers=(((1,), (0,)), ((), ())),
            preferred_element_type=jnp.float32,
        )
        best = v[0:16, :]
        code = jnp.zeros(best.shape, dtype=jnp.float32)
        for k in range(1, 8):
            s = v[16 * k:16 * (k + 1), :]
            gt = s > best
            code = jnp.where(gt, jnp.float32(k), code)
            best = jnp.maximum(best, s)
        codep = jax.lax.dot_general(
            code.astype(jnp.bfloat16), perm_ref[...],
            dimension_numbers=(((1,), (0,)), ((), ())),
            preferred_element_type=jnp.float32,
        )
        stacked = jnp.concatenate(
            [codep[:, p * tg:(p + 1) * tg] for p in range(8)], axis=0
        )
        ci = stacked.T.astype(jnp.int32)
        packed = jnp.full(ci.shape, _PACKED_LUT[0], dtype=jnp.int32)
        for k in range(1, 8):
            packed = jnp.where(ci == k, _PACKED_LUT[k], packed)
        rows = pl.ds(row * tg, tg)
        r_ref[rows, :] = (packed & 0xFF).astype(jnp.uint8)
        g_ref[rows, :] = ((packed >> 8) & 0xFF).astype(jnp.uint8)
        b_ref[rows, :] = ((packed >> 16) & 0xFF).astype(jnp.uint8)


def kernel(inp, rotations):
    B, C, H, W = inp.shape
    HW = H * W
    n_hashes = rotations.shape[1]
    rot = jnp.transpose(rotations, (0, 2, 1)).reshape(C, 4 * n_hashes)
    rot_packed = jnp.concatenate([rot, -rot], axis=1).T

    tg = W // 8
    j = jnp.arange(W, dtype=jnp.int32)
    c_of_j = 8 * (j % tg) + j // tg
    perm = (jnp.arange(W, dtype=jnp.int32)[:, None] == c_of_j[None, :]
            ).astype(jnp.bfloat16)

    out_sds = jax.ShapeDtypeStruct((HW // 8, 128), jnp.uint8)
    blk_rows = _HBLK * tg
    r, g, b = pl.pallas_call(
        _lsh_kernel,
        grid=(H // _HBLK,),
        in_specs=[
            pl.BlockSpec((128, C), lambda i: (0, 0)),
            pl.BlockSpec((1, C, _HBLK, W), lambda i: (0, 0, i, 0)),
            pl.BlockSpec((W, W), lambda i: (0, 0)),
        ],
        out_specs=[pl.BlockSpec((blk_rows, 128), lambda i: (i, 0))] * 3,
        out_shape=[out_sds, out_sds, out_sds],
    )(rot_packed, inp, perm)
    shape = (B, H, W, n_hashes)
    return (r.reshape(shape), g.reshape(shape), b.reshape(shape))


# speedup vs baseline: 2.2548x; 1.5875x over previous
import jax
import jax.numpy as jnp
from jax.experimental import pallas as pl

_R = (0, 46, 167, 100, 191, 220, 0, 10)
_G = (160, 141, 0, 62, 30, 87, 166, 91)
_B = (177, 239, 174, 191, 75, 46, 0, 196)
_PACKED_LUT = tuple(r | (g << 8) | (b << 16) for r, g, b in zip(_R, _G, _B))

_HBLK = 8


def _lsh_kernel(rot_ref, x_ref, r_ref, g_ref, b_ref):
    for row in range(x_ref.shape[2]):
        v = jax.lax.dot_general(
            rot_ref[...], x_ref[0, :, row, :],
            dimension_numbers=(((1,), (0,)), ((), ())),
            preferred_element_type=jnp.float32,
        )
        best = v[0:16, :]
        packed = jnp.full(best.shape, _PACKED_LUT[0], dtype=jnp.int32)
        for k in range(1, 8):
            s = v[16 * k:16 * (k + 1), :]
            gt = s > best
            packed = jnp.where(gt, _PACKED_LUT[k], packed)
            best = jnp.maximum(best, s)
        r_ref[0, row] = (packed & 0xFF).astype(jnp.uint8).T
        g_ref[0, row] = ((packed >> 8) & 0xFF).astype(jnp.uint8).T
        b_ref[0, row] = ((packed >> 16) & 0xFF).astype(jnp.uint8).T


def kernel(inp, rotations):
    B, C, H, W = inp.shape
    n_hashes = rotations.shape[1]
    rot = jnp.transpose(rotations, (0, 2, 1)).reshape(C, 4 * n_hashes)
    rot_packed = jnp.concatenate([rot, -rot], axis=1).T

    out_sds = jax.ShapeDtypeStruct((B, H, W, n_hashes), jnp.uint8)
    return pl.pallas_call(
        _lsh_kernel,
        grid=(H // _HBLK,),
        in_specs=[
            pl.BlockSpec((128, C), lambda i: (0, 0)),
            pl.BlockSpec((1, C, _HBLK, W), lambda i: (0, 0, i, 0)),
        ],
        out_specs=[pl.BlockSpec((1, _HBLK, W, n_hashes), lambda i: (0, i, 0, 0))] * 3,
        out_shape=[out_sds, out_sds, out_sds],
    )(rot_packed, inp)


# speedup vs baseline: 2.3249x; 1.0311x over previous
import jax
import jax.numpy as jnp
from jax.experimental import pallas as pl

_R = (0, 46, 167, 100, 191, 220, 0, 10)
_G = (160, 141, 0, 62, 30, 87, 166, 91)
_B = (177, 239, 174, 191, 75, 46, 0, 196)
_PACKED_LUT = tuple(r | (g << 8) | (b << 16) for r, g, b in zip(_R, _G, _B))

_HBLK = 16


def _lsh_kernel(rot_ref, x_ref, r_ref, g_ref, b_ref):
    for row in range(x_ref.shape[2]):
        v = jax.lax.dot_general(
            rot_ref[...], x_ref[0, :, row, :],
            dimension_numbers=(((1,), (0,)), ((), ())),
            preferred_element_type=jnp.float32,
        )
        best = v[0:16, :]
        packed = jnp.full(best.shape, _PACKED_LUT[0], dtype=jnp.int32)
        for k in range(1, 8):
            s = v[16 * k:16 * (k + 1), :]
            gt = s > best
            packed = jnp.where(gt, _PACKED_LUT[k], packed)
            best = jnp.maximum(best, s)
        r_ref[0, row] = (packed & 0xFF).astype(jnp.uint8).T
        g_ref[0, row] = ((packed >> 8) & 0xFF).astype(jnp.uint8).T
        b_ref[0, row] = ((packed >> 16) & 0xFF).astype(jnp.uint8).T


def kernel(inp, rotations):
    B, C, H, W = inp.shape
    n_hashes = rotations.shape[1]
    rot = jnp.transpose(rotations, (0, 2, 1)).reshape(C, 4 * n_hashes)
    rot_packed = jnp.concatenate([rot, -rot], axis=1).T

    out_sds = jax.ShapeDtypeStruct((B, H, W, n_hashes), jnp.uint8)
    return pl.pallas_call(
        _lsh_kernel,
        grid=(H // _HBLK,),
        in_specs=[
            pl.BlockSpec((128, C), lambda i: (0, 0)),
            pl.BlockSpec((1, C, _HBLK, W), lambda i: (0, 0, i, 0)),
        ],
        out_specs=[pl.BlockSpec((1, _HBLK, W, n_hashes), lambda i: (0, i, 0, 0))] * 3,
        out_shape=[out_sds, out_sds, out_sds],
    )(rot_packed, inp)
